# baseline (speedup 1.0000x reference)
import functools

import jax
import jax.numpy as jnp
from jax import lax
from jax.experimental import pallas as pl
from jax.experimental.pallas import tpu as pltpu
from jax.experimental.pallas import tpu_sc as plsc

NC = 2
NS = 16
NW = NC * NS
LANE = 128
NUM_GRAPHS = 64


def _mesh():
    return plsc.VectorSubcoreMesh(core_axis_name="c", subcore_axis_name="s")


@functools.lru_cache(maxsize=None)
def _make_deg(npad, nblk, stripe):
    @functools.partial(
        pl.kernel,
        mesh=_mesh(),
        out_type=[
            jax.ShapeDtypeStruct((npad,), jnp.float32),
            jax.ShapeDtypeStruct((npad,), jnp.float32),
        ],
        scratch_types=[
            pltpu.VMEM((nblk, LANE), jnp.int32),
            pltpu.VMEM((LANE,), jnp.float32),
            pltpu.VMEM_SHARED((npad,), jnp.float32),
        ],
    )
    def deg_kernel(dst_hbm, zeros_hbm, out0, out1, idx_v, ones_v, acc_sh):
        c = lax.axis_index("c")
        s = lax.axis_index("s")
        wid = s * NC + c
        seg = pl.ds(s * stripe, stripe)
        pltpu.sync_copy(zeros_hbm, acc_sh.at[seg])
        pltpu.sync_copy(dst_hbm.at[wid], idx_v)
        for i in range(LANE // 16):
            ones_v[pl.ds(i * 16, 16)] = jnp.full((16,), 1.0, jnp.float32)
        plsc.subcore_barrier()

        def body(j, carry):
            pltpu.sync_copy(ones_v, acc_sh.at[idx_v.at[j]], add=True)
            return carry

        lax.fori_loop(0, nblk, body, 0)
        plsc.subcore_barrier()

        @pl.when(c == 0)
        def _():
            pltpu.sync_copy(acc_sh.at[seg], out0.at[seg])

        @pl.when(c == 1)
        def _():
            pltpu.sync_copy(acc_sh.at[seg], out1.at[seg])

    return deg_kernel


KSB = 16


@functools.lru_cache(maxsize=None)
def _make_agg(npad, nblk, stripe, d):
    assert nblk % 2 == 0 and nblk % KSB == 0 and KSB % 2 == 0
    nsb = nblk // KSB

    @functools.partial(
        pl.kernel,
        mesh=_mesh(),
        out_type=[
            jax.ShapeDtypeStruct((npad, d), jnp.float32),
            jax.ShapeDtypeStruct((npad, d), jnp.float32),
        ],
        scratch_types=[
            pltpu.VMEM((KSB, LANE), jnp.int32),
            pltpu.VMEM((KSB, LANE), jnp.int32),
            pltpu.VMEM((KSB, LANE), jnp.int32),
            pltpu.VMEM((KSB, LANE), jnp.int32),
            pltpu.VMEM((LANE, d), jnp.float32),
            pltpu.VMEM((LANE, d), jnp.float32),
            pltpu.VMEM_SHARED((npad, d), jnp.float32),
            pltpu.SemaphoreType.DMA,
            pltpu.SemaphoreType.DMA,
            pltpu.SemaphoreType.DMA,
            pltpu.SemaphoreType.DMA,
            pltpu.SemaphoreType.DMA,
        ],
    )
    def agg_kernel(y_hbm, src_hbm, dst_hbm, zrows_hbm, out0, out1,
                   si0, di0, si1, di1, rows_a, rows_b, acc_sh,
                   sem_ga, sem_gb, sem_sa, sem_sb, sem_i):
        c = lax.axis_index("c")
        s = lax.axis_index("s")
        wid = s * NC + c
        seg = pl.ds(s * stripe, stripe)
        pltpu.sync_copy(src_hbm.at[wid, pl.ds(0, KSB)], si0)
        pltpu.sync_copy(dst_hbm.at[wid, pl.ds(0, KSB)], di0)
        pltpu.sync_copy(zrows_hbm, acc_sh.at[seg])
        plsc.subcore_barrier()

        idx_bufs = [(si0, di0), (si1, di1)]
        for sb in range(nsb):
            si, di = idx_bufs[sb % 2]
            sn, dn = idx_bufs[(sb + 1) % 2]
            if sb > 0:
                pltpu.make_async_copy(
                    src_hbm.at[wid, pl.ds(sb * KSB, KSB)], si, sem_i).wait()
                pltpu.make_async_copy(
                    dst_hbm.at[wid, pl.ds(sb * KSB, KSB)], di, sem_i).wait()
            if sb + 1 < nsb:
                nxt = pl.ds((sb + 1) * KSB, KSB)
                pltpu.async_copy(src_hbm.at[wid, nxt], sn, sem_i)
                pltpu.async_copy(dst_hbm.at[wid, nxt], dn, sem_i)
            def gat(si, j, buf, sem):
                h = LANE // 2
                pltpu.async_copy(
                    y_hbm.at[si.at[j, pl.ds(0, h)]], buf.at[pl.ds(0, h)], sem)
                pltpu.async_copy(
                    y_hbm.at[si.at[j, pl.ds(h, h)]], buf.at[pl.ds(h, h)], sem)

            gat(si, 0, rows_a, sem_ga)
            gat(si, 1, rows_b, sem_gb)

            def body(p, carry, si=si, di=di):
                j = p * 2
                pltpu.make_async_copy(
                    y_hbm.at[si.at[j]], rows_a, sem_ga).wait()
                pltpu.async_copy(rows_a, acc_sh.at[di.at[j]], sem_sa,
                                 add=True)
                pltpu.make_async_copy(
                    y_hbm.at[si.at[j + 1]], rows_b, sem_gb).wait()
                pltpu.async_copy(rows_b, acc_sh.at[di.at[j + 1]], sem_sb,
                                 add=True)
                pltpu.make_async_copy(
                    rows_a, acc_sh.at[di.at[j]], sem_sa).wait()

                @pl.when(j + 2 < KSB)
                def _():
                    gat(si, j + 2, rows_a, sem_ga)

                pltpu.make_async_copy(
                    rows_b, acc_sh.at[di.at[j + 1]], sem_sb).wait()

                @pl.when(j + 3 < KSB)
                def _():
                    gat(si, j + 3, rows_b, sem_gb)

                return carry

            lax.fori_loop(0, KSB // 2, body, 0)
        plsc.subcore_barrier()

        @pl.when(c == 0)
        def _():
            pltpu.sync_copy(acc_sh.at[seg], out0.at[seg])

        @pl.when(c == 1)
        def _():
            pltpu.sync_copy(acc_sh.at[seg], out1.at[seg])

    return agg_kernel


def _prep_tc(x, w1, d0, d1):
    n = x.shape[0]
    d = w1.shape[1]

    def body(x_ref, w_ref, d0_ref, d1_ref, y_ref, dis_ref):
        deg = d0_ref[...][:n] + d1_ref[...][:n] + 1.0
        dis = lax.rsqrt(deg)
        dis_ref[...] = dis
        xw = jnp.dot(x_ref[...], w_ref[...], preferred_element_type=jnp.float32)
        y_ref[...] = xw * dis

    return pl.pallas_call(
        body,
        out_shape=[
            jax.ShapeDtypeStruct((n, d), jnp.float32),
            jax.ShapeDtypeStruct((n, 1), jnp.float32),
        ],
    )(x, w1, d0, d1)


def _mid_tc(p0, p1, y1, dis, b1, w2):
    n, d = y1.shape

    def body(p0_ref, p1_ref, y1_ref, dis_ref, b_ref, w_ref, y2_ref):
        acc = p0_ref[...][:n] + p1_ref[...][:n] + y1_ref[...]
        h = jnp.maximum(dis_ref[...] * acc + b_ref[...], 0.0)
        hw = jnp.dot(h, w_ref[...], preferred_element_type=jnp.float32)
        y2_ref[...] = hw * dis_ref[...]

    return pl.pallas_call(
        body,
        out_shape=jax.ShapeDtypeStruct((n, d), jnp.float32),
    )(p0, p1, y1, dis, b1, w2)


def _head_tc(q0, q1, y2, dis, b2, batch_row, fc_w, fc_b):
    n, d = y2.shape
    d_out = fc_w.shape[1]

    def body(q0_ref, q1_ref, y2_ref, dis_ref, b_ref, batch_ref, fw_ref,
             fb_ref, out_ref):
        acc = q0_ref[...][:n] + q1_ref[...][:n] + y2_ref[...]
        h = jnp.maximum(dis_ref[...] * acc + b_ref[...], 0.0)
        gids = lax.broadcasted_iota(jnp.int32, (NUM_GRAPHS, n), 0)
        m = (gids == batch_ref[...]).astype(jnp.float32)
        sums = jnp.dot(m, h, preferred_element_type=jnp.float32)
        counts = jnp.sum(m, axis=1, keepdims=True)
        pooled = sums / jnp.maximum(counts, 1.0)
        out_ref[...] = (
            jnp.dot(pooled, fw_ref[...], preferred_element_type=jnp.float32)
            + fb_ref[...]
        )

    return pl.pallas_call(
        body,
        out_shape=jax.ShapeDtypeStruct((NUM_GRAPHS, d_out), jnp.float32),
    )(q0, q1, y2, dis, b2, batch_row, fc_w, fc_b)


def kernel(x, edge_index, batch, W1, b1, W2, b2, fc_w, fc_b):
    n, _ = x.shape
    d = W1.shape[1]
    e = edge_index.shape[1]

    npad = -(-(n + 8) // (NS * LANE)) * (NS * LANE)
    stripe = npad // NS
    nblk = -(-e // (NW * LANE))
    nblk = -(-nblk // KSB) * KSB
    pade = NW * nblk * LANE

    ei = edge_index.astype(jnp.int32)
    pad = pade - e
    pad_ar = jnp.arange(pad, dtype=jnp.int32)
    src_p = jnp.concatenate(
        [ei[0], pad_ar % n]).reshape(NW, nblk, LANE)
    dst_p = jnp.concatenate(
        [ei[1], n + pad_ar % (npad - n)]).reshape(NW, nblk, LANE)

    z1 = jnp.zeros((stripe,), jnp.float32)
    z2 = jnp.zeros((stripe, d), jnp.float32)

    deg0, deg1 = _make_deg(npad, nblk, stripe)(dst_p, z1)
    y1, dis = _prep_tc(x, W1, deg0.reshape(npad, 1), deg1.reshape(npad, 1))

    agg = _make_agg(npad, nblk, stripe, d)
    p0, p1 = agg(y1, src_p, dst_p, z2)
    y2 = _mid_tc(p0, p1, y1, dis, b1.reshape(1, d), W2)
    q0, q1 = agg(y2, src_p, dst_p, z2)

    out = _head_tc(q0, q1, y2, dis, b2.reshape(1, d),
                   batch.astype(jnp.int32).reshape(1, n),
                   fc_w, fc_b.reshape(1, fc_w.shape[1]))
    return out


# speedup vs baseline: 1.0038x; 1.0038x over previous
import functools

import jax
import jax.numpy as jnp
from jax import lax
from jax.experimental import pallas as pl
from jax.experimental.pallas import tpu as pltpu
from jax.experimental.pallas import tpu_sc as plsc

NC = 2
NS = 16
NW = NC * NS
LANE = 128
NUM_GRAPHS = 64


def _mesh():
    return plsc.VectorSubcoreMesh(core_axis_name="c", subcore_axis_name="s")


@functools.lru_cache(maxsize=None)
def _make_deg(npad, nblk, stripe):
    @functools.partial(
        pl.kernel,
        mesh=_mesh(),
        out_type=[
            jax.ShapeDtypeStruct((npad,), jnp.float32),
            jax.ShapeDtypeStruct((npad,), jnp.float32),
        ],
        scratch_types=[
            pltpu.VMEM((nblk, LANE), jnp.int32),
            pltpu.VMEM((LANE,), jnp.float32),
            pltpu.VMEM_SHARED((npad,), jnp.float32),
        ],
    )
    def deg_kernel(dst_hbm, zeros_hbm, out0, out1, idx_v, ones_v, acc_sh):
        c = lax.axis_index("c")
        s = lax.axis_index("s")
        wid = s * NC + c
        seg = pl.ds(s * stripe, stripe)
        pltpu.sync_copy(zeros_hbm, acc_sh.at[seg])
        pltpu.sync_copy(dst_hbm.at[wid], idx_v)
        for i in range(LANE // 16):
            ones_v[pl.ds(i * 16, 16)] = jnp.full((16,), 1.0, jnp.float32)
        plsc.subcore_barrier()

        def body(j, carry):
            pltpu.sync_copy(ones_v, acc_sh.at[idx_v.at[j]], add=True)
            return carry

        lax.fori_loop(0, nblk, body, 0)
        plsc.subcore_barrier()

        @pl.when(c == 0)
        def _():
            pltpu.sync_copy(acc_sh.at[seg], out0.at[seg])

        @pl.when(c == 1)
        def _():
            pltpu.sync_copy(acc_sh.at[seg], out1.at[seg])

    return deg_kernel


KSB = 16


@functools.lru_cache(maxsize=None)
def _make_agg(npad, nblk, stripe, d):
    assert nblk % 2 == 0 and nblk % KSB == 0 and KSB % 2 == 0
    nsb = nblk // KSB

    @functools.partial(
        pl.kernel,
        mesh=_mesh(),
        out_type=[
            jax.ShapeDtypeStruct((npad, d), jnp.float32),
            jax.ShapeDtypeStruct((npad, d), jnp.float32),
        ],
        scratch_types=[
            pltpu.VMEM((KSB, LANE), jnp.int32),
            pltpu.VMEM((KSB, LANE), jnp.int32),
            pltpu.VMEM((KSB, LANE), jnp.int32),
            pltpu.VMEM((KSB, LANE), jnp.int32),
            pltpu.VMEM((LANE, d), jnp.float32),
            pltpu.VMEM((LANE, d), jnp.float32),
            pltpu.VMEM_SHARED((npad, d), jnp.float32),
            pltpu.SemaphoreType.DMA,
            pltpu.SemaphoreType.DMA,
            pltpu.SemaphoreType.DMA,
            pltpu.SemaphoreType.DMA,
            pltpu.SemaphoreType.DMA,
        ],
    )
    def agg_kernel(y_hbm, src_hbm, dst_hbm, zrows_hbm, out0, out1,
                   si0, di0, si1, di1, rows_a, rows_b, acc_sh,
                   sem_ga, sem_gb, sem_sa, sem_sb, sem_i):
        c = lax.axis_index("c")
        s = lax.axis_index("s")
        wid = s * NC + c
        seg = pl.ds(s * stripe, stripe)
        pltpu.sync_copy(src_hbm.at[wid, pl.ds(0, KSB)], si0)
        pltpu.sync_copy(dst_hbm.at[wid, pl.ds(0, KSB)], di0)
        pltpu.sync_copy(zrows_hbm, acc_sh.at[seg])
        plsc.subcore_barrier()

        idx_bufs = [(si0, di0), (si1, di1)]
        for sb in range(nsb):
            si, di = idx_bufs[sb % 2]
            sn, dn = idx_bufs[(sb + 1) % 2]
            if sb > 0:
                pltpu.make_async_copy(
                    src_hbm.at[wid, pl.ds(sb * KSB, KSB)], si, sem_i).wait()
                pltpu.make_async_copy(
                    dst_hbm.at[wid, pl.ds(sb * KSB, KSB)], di, sem_i).wait()
            if sb + 1 < nsb:
                nxt = pl.ds((sb + 1) * KSB, KSB)
                pltpu.async_copy(src_hbm.at[wid, nxt], sn, sem_i)
                pltpu.async_copy(dst_hbm.at[wid, nxt], dn, sem_i)
            pltpu.async_copy(y_hbm.at[si.at[0]], rows_a, sem_ga)
            pltpu.async_copy(y_hbm.at[si.at[1]], rows_b, sem_gb)

            def body(p, carry, si=si, di=di):
                j = p * 2
                pltpu.make_async_copy(
                    y_hbm.at[si.at[j]], rows_a, sem_ga).wait()
                pltpu.async_copy(rows_a, acc_sh.at[di.at[j]], sem_sa,
                                 add=True)
                pltpu.make_async_copy(
                    y_hbm.at[si.at[j + 1]], rows_b, sem_gb).wait()
                pltpu.async_copy(rows_b, acc_sh.at[di.at[j + 1]], sem_sb,
                                 add=True)
                pltpu.make_async_copy(
                    rows_a, acc_sh.at[di.at[j]], sem_sa).wait()

                @pl.when(j + 2 < KSB)
                def _():
                    pltpu.async_copy(y_hbm.at[si.at[j + 2]], rows_a, sem_ga)

                pltpu.make_async_copy(
                    rows_b, acc_sh.at[di.at[j + 1]], sem_sb).wait()

                @pl.when(j + 3 < KSB)
                def _():
                    pltpu.async_copy(y_hbm.at[si.at[j + 3]], rows_b, sem_gb)

                return carry

            lax.fori_loop(0, KSB // 2, body, 0)
        plsc.subcore_barrier()

        @pl.when(c == 0)
        def _():
            pltpu.sync_copy(acc_sh.at[seg], out0.at[seg])

        @pl.when(c == 1)
        def _():
            pltpu.sync_copy(acc_sh.at[seg], out1.at[seg])

    return agg_kernel


def _xw_tc(x, w1):
    n = x.shape[0]
    d = w1.shape[1]

    def body(x_ref, w_ref, xw_ref):
        xw_ref[...] = jnp.dot(x_ref[...], w_ref[...],
                              preferred_element_type=jnp.float32)

    return pl.pallas_call(
        body,
        out_shape=jax.ShapeDtypeStruct((n, d), jnp.float32),
    )(x, w1)


def _scale_tc(xw, d0, d1):
    n, d = xw.shape

    def body(xw_ref, d0_ref, d1_ref, y_ref, dis_ref):
        deg = d0_ref[...][:n] + d1_ref[...][:n] + 1.0
        dis = lax.rsqrt(deg)
        dis_ref[...] = dis
        y_ref[...] = xw_ref[...] * dis

    return pl.pallas_call(
        body,
        out_shape=[
            jax.ShapeDtypeStruct((n, d), jnp.float32),
            jax.ShapeDtypeStruct((n, 1), jnp.float32),
        ],
    )(xw, d0, d1)


def _mid_tc(p0, p1, y1, dis, b1, w2):
    n, d = y1.shape

    def body(p0_ref, p1_ref, y1_ref, dis_ref, b_ref, w_ref, y2_ref):
        acc = p0_ref[...][:n] + p1_ref[...][:n] + y1_ref[...]
        h = jnp.maximum(dis_ref[...] * acc + b_ref[...], 0.0)
        hw = jnp.dot(h, w_ref[...], preferred_element_type=jnp.float32)
        y2_ref[...] = hw * dis_ref[...]

    return pl.pallas_call(
        body,
        out_shape=jax.ShapeDtypeStruct((n, d), jnp.float32),
    )(p0, p1, y1, dis, b1, w2)


def _head_tc(q0, q1, y2, dis, b2, batch_row, fc_w, fc_b):
    n, d = y2.shape
    d_out = fc_w.shape[1]

    def body(q0_ref, q1_ref, y2_ref, dis_ref, b_ref, batch_ref, fw_ref,
             fb_ref, out_ref):
        acc = q0_ref[...][:n] + q1_ref[...][:n] + y2_ref[...]
        h = jnp.maximum(dis_ref[...] * acc + b_ref[...], 0.0)
        gids = lax.broadcasted_iota(jnp.int32, (NUM_GRAPHS, n), 0)
        m = (gids == batch_ref[...]).astype(jnp.float32)
        sums = jnp.dot(m, h, preferred_element_type=jnp.float32)
        counts = jnp.sum(m, axis=1, keepdims=True)
        pooled = sums / jnp.maximum(counts, 1.0)
        out_ref[...] = (
            jnp.dot(pooled, fw_ref[...], preferred_element_type=jnp.float32)
            + fb_ref[...]
        )

    return pl.pallas_call(
        body,
        out_shape=jax.ShapeDtypeStruct((NUM_GRAPHS, d_out), jnp.float32),
    )(q0, q1, y2, dis, b2, batch_row, fc_w, fc_b)


def kernel(x, edge_index, batch, W1, b1, W2, b2, fc_w, fc_b):
    n, _ = x.shape
    d = W1.shape[1]
    e = edge_index.shape[1]

    npad = -(-(n + 8) // (NS * LANE)) * (NS * LANE)
    stripe = npad // NS
    nblk = -(-e // (NW * LANE))
    nblk = -(-nblk // KSB) * KSB
    pade = NW * nblk * LANE

    ei = edge_index.astype(jnp.int32)
    pad = pade - e
    pad_ar = jnp.arange(pad, dtype=jnp.int32)
    src_p = jnp.concatenate(
        [ei[0], pad_ar % n]).reshape(NW, nblk, LANE)
    dst_p = jnp.concatenate(
        [ei[1], n + pad_ar % (npad - n)]).reshape(NW, nblk, LANE)

    z1 = jnp.zeros((stripe,), jnp.float32)
    z2 = jnp.zeros((stripe, d), jnp.float32)

    deg0, deg1 = _make_deg(npad, nblk, stripe)(dst_p, z1)
    xw = _xw_tc(x, W1)
    y1, dis = _scale_tc(xw, deg0.reshape(npad, 1), deg1.reshape(npad, 1))

    agg = _make_agg(npad, nblk, stripe, d)
    p0, p1 = agg(y1, src_p, dst_p, z2)
    y2 = _mid_tc(p0, p1, y1, dis, b1.reshape(1, d), W2)
    q0, q1 = agg(y2, src_p, dst_p, z2)

    out = _head_tc(q0, q1, y2, dis, b2.reshape(1, d),
                   batch.astype(jnp.int32).reshape(1, n),
                   fc_w, fc_b.reshape(1, fc_w.shape[1]))
    return out


# speedup vs baseline: 1.0085x; 1.0046x over previous
import functools

import jax
import jax.numpy as jnp
from jax import lax
from jax.experimental import pallas as pl
from jax.experimental.pallas import tpu as pltpu
from jax.experimental.pallas import tpu_sc as plsc

NC = 2
NS = 16
NW = NC * NS
LANE = 128
NUM_GRAPHS = 64


def _mesh():
    return plsc.VectorSubcoreMesh(core_axis_name="c", subcore_axis_name="s")


@functools.lru_cache(maxsize=None)
def _make_deg(npad, nblk, stripe):
    @functools.partial(
        pl.kernel,
        mesh=_mesh(),
        out_type=[
            jax.ShapeDtypeStruct((npad,), jnp.float32),
            jax.ShapeDtypeStruct((npad,), jnp.float32),
        ],
        scratch_types=[
            pltpu.VMEM((nblk, LANE), jnp.int32),
            pltpu.VMEM((LANE,), jnp.float32),
            pltpu.VMEM_SHARED((npad,), jnp.float32),
        ],
    )
    def deg_kernel(dst_hbm, zeros_hbm, out0, out1, idx_v, ones_v, acc_sh):
        c = lax.axis_index("c")
        s = lax.axis_index("s")
        wid = s * NC + c
        seg = pl.ds(s * stripe, stripe)
        pltpu.sync_copy(zeros_hbm, acc_sh.at[seg])
        pltpu.sync_copy(dst_hbm.at[wid], idx_v)
        for i in range(LANE // 16):
            ones_v[pl.ds(i * 16, 16)] = jnp.full((16,), 1.0, jnp.float32)
        plsc.subcore_barrier()

        def body(j, carry):
            pltpu.sync_copy(ones_v, acc_sh.at[idx_v.at[j]], add=True)
            return carry

        lax.fori_loop(0, nblk, body, 0)
        plsc.subcore_barrier()

        @pl.when(c == 0)
        def _():
            pltpu.sync_copy(acc_sh.at[seg], out0.at[seg])

        @pl.when(c == 1)
        def _():
            pltpu.sync_copy(acc_sh.at[seg], out1.at[seg])

    return deg_kernel


KSB = 16


@functools.lru_cache(maxsize=None)
def _make_agg(npad, nblk, stripe, d):
    assert nblk % 2 == 0 and nblk % KSB == 0 and KSB % 2 == 0
    nsb = nblk // KSB

    @functools.partial(
        pl.kernel,
        mesh=_mesh(),
        out_type=[
            jax.ShapeDtypeStruct((npad, d), jnp.float32),
            jax.ShapeDtypeStruct((npad, d), jnp.float32),
        ],
        scratch_types=[
            pltpu.VMEM((KSB, LANE), jnp.int32),
            pltpu.VMEM((KSB, LANE), jnp.int32),
            pltpu.VMEM((KSB, LANE), jnp.int32),
            pltpu.VMEM((KSB, LANE), jnp.int32),
            pltpu.VMEM((LANE, d), jnp.float32),
            pltpu.VMEM((LANE, d), jnp.float32),
            pltpu.VMEM_SHARED((npad, d), jnp.float32),
            pltpu.SemaphoreType.DMA,
            pltpu.SemaphoreType.DMA,
            pltpu.SemaphoreType.DMA,
            pltpu.SemaphoreType.DMA,
            pltpu.SemaphoreType.DMA,
        ],
    )
    def agg_kernel(y_hbm, src_hbm, dst_hbm, zrows_hbm, out0, out1,
                   si0, di0, si1, di1, rows_a, rows_b, acc_sh,
                   sem_ga, sem_gb, sem_sa, sem_sb, sem_i):
        c = lax.axis_index("c")
        s = lax.axis_index("s")
        wid = s * NC + c
        seg = pl.ds(s * stripe, stripe)
        pltpu.sync_copy(src_hbm.at[wid, pl.ds(0, KSB)], si0)
        pltpu.sync_copy(dst_hbm.at[wid, pl.ds(0, KSB)], di0)
        pltpu.sync_copy(zrows_hbm, acc_sh.at[seg])
        plsc.subcore_barrier()

        idx_bufs = [(si0, di0), (si1, di1)]
        for sb in range(nsb):
            si, di = idx_bufs[sb % 2]
            sn, dn = idx_bufs[(sb + 1) % 2]
            if sb > 0:
                pltpu.make_async_copy(
                    src_hbm.at[wid, pl.ds(sb * KSB, KSB)], si, sem_i).wait()
                pltpu.make_async_copy(
                    dst_hbm.at[wid, pl.ds(sb * KSB, KSB)], di, sem_i).wait()
            if sb + 1 < nsb:
                nxt = pl.ds((sb + 1) * KSB, KSB)
                pltpu.async_copy(src_hbm.at[wid, nxt], sn, sem_i)
                pltpu.async_copy(dst_hbm.at[wid, nxt], dn, sem_i)
            pltpu.async_copy(y_hbm.at[si.at[0]], rows_a, sem_ga)
            pltpu.async_copy(y_hbm.at[si.at[1]], rows_b, sem_gb)

            def body(p, carry, si=si, di=di):
                j = p * 2
                pltpu.make_async_copy(
                    y_hbm.at[si.at[j]], rows_a, sem_ga).wait()
                pltpu.async_copy(rows_a, acc_sh.at[di.at[j]], sem_sa,
                                 add=True)
                pltpu.make_async_copy(
                    y_hbm.at[si.at[j + 1]], rows_b, sem_gb).wait()
                pltpu.async_copy(rows_b, acc_sh.at[di.at[j + 1]], sem_sb,
                                 add=True)
                pltpu.make_async_copy(
                    rows_a, acc_sh.at[di.at[j]], sem_sa).wait()

                @pl.when(j + 2 < KSB)
                def _():
                    pltpu.async_copy(y_hbm.at[si.at[j + 2]], rows_a, sem_ga)

                pltpu.make_async_copy(
                    rows_b, acc_sh.at[di.at[j + 1]], sem_sb).wait()

                @pl.when(j + 3 < KSB)
                def _():
                    pltpu.async_copy(y_hbm.at[si.at[j + 3]], rows_b, sem_gb)

                return carry

            lax.fori_loop(0, KSB // 2, body, 0)
        plsc.subcore_barrier()

        @pl.when(c == 0)
        def _():
            pltpu.sync_copy(acc_sh.at[seg], out0.at[seg])

        @pl.when(c == 1)
        def _():
            pltpu.sync_copy(acc_sh.at[seg], out1.at[seg])

    return agg_kernel


def _prep_tc(x, w1, d0, d1):
    n = x.shape[0]
    d = w1.shape[1]

    def body(x_ref, w_ref, d0_ref, d1_ref, y_ref, dis_ref):
        deg = d0_ref[...][:n] + d1_ref[...][:n] + 1.0
        dis = lax.rsqrt(deg)
        dis_ref[...] = dis
        xw = jnp.dot(x_ref[...], w_ref[...], preferred_element_type=jnp.float32)
        y_ref[...] = xw * dis

    return pl.pallas_call(
        body,
        out_shape=[
            jax.ShapeDtypeStruct((n, d), jnp.float32),
            jax.ShapeDtypeStruct((n, 1), jnp.float32),
        ],
    )(x, w1, d0, d1)


def _mid_tc(p0, p1, y1, dis, b1, w2):
    n, d = y1.shape

    def body(p0_ref, p1_ref, y1_ref, dis_ref, b_ref, w_ref, y2_ref):
        acc = p0_ref[...][:n] + p1_ref[...][:n] + y1_ref[...]
        h = jnp.maximum(dis_ref[...] * acc + b_ref[...], 0.0)
        hw = jnp.dot(h, w_ref[...], preferred_element_type=jnp.float32)
        y2_ref[...] = hw * dis_ref[...]

    return pl.pallas_call(
        body,
        out_shape=jax.ShapeDtypeStruct((n, d), jnp.float32),
    )(p0, p1, y1, dis, b1, w2)


def _head_tc(q0, q1, y2, dis, b2, batch_row, fc_w, fc_b):
    n, d = y2.shape
    d_out = fc_w.shape[1]

    def body(q0_ref, q1_ref, y2_ref, dis_ref, b_ref, batch_ref, fw_ref,
             fb_ref, out_ref):
        acc = q0_ref[...][:n] + q1_ref[...][:n] + y2_ref[...]
        h = jnp.maximum(dis_ref[...] * acc + b_ref[...], 0.0)
        gids = lax.broadcasted_iota(jnp.int32, (NUM_GRAPHS, n), 0)
        m = (gids == batch_ref[...]).astype(jnp.float32)
        sums = jnp.dot(m, h, preferred_element_type=jnp.float32)
        counts = jnp.sum(m, axis=1, keepdims=True)
        pooled = sums / jnp.maximum(counts, 1.0)
        out_ref[...] = (
            jnp.dot(pooled, fw_ref[...], preferred_element_type=jnp.float32)
            + fb_ref[...]
        )

    return pl.pallas_call(
        body,
        out_shape=jax.ShapeDtypeStruct((NUM_GRAPHS, d_out), jnp.float32),
    )(q0, q1, y2, dis, b2, batch_row, fc_w, fc_b)


def kernel(x, edge_index, batch, W1, b1, W2, b2, fc_w, fc_b):
    n, _ = x.shape
    d = W1.shape[1]
    e = edge_index.shape[1]

    npad = -(-(n + 8) // (NS * LANE)) * (NS * LANE)
    stripe = npad // NS
    nblk = -(-e // (NW * LANE))
    nblk = -(-nblk // KSB) * KSB
    pade = NW * nblk * LANE

    ei = edge_index.astype(jnp.int32)
    pad = pade - e
    pad_ar = jnp.arange(pad, dtype=jnp.int32)
    src_p = jnp.concatenate(
        [ei[0], pad_ar % n]).reshape(NW, nblk, LANE)
    dst_p = jnp.concatenate(
        [ei[1], n + pad_ar % (npad - n)]).reshape(NW, nblk, LANE)

    z1 = jnp.zeros((stripe,), jnp.float32)
    z2 = jnp.zeros((stripe, d), jnp.float32)

    deg0, deg1 = _make_deg(npad, nblk, stripe)(dst_p, z1)
    y1, dis = _prep_tc(x, W1, deg0.reshape(npad, 1), deg1.reshape(npad, 1))

    agg = _make_agg(npad, nblk, stripe, d)
    p0, p1 = agg(y1, src_p, dst_p, z2)
    y2 = _mid_tc(p0, p1, y1, dis, b1.reshape(1, d), W2)
    q0, q1 = agg(y2, src_p, dst_p, z2)

    out = _head_tc(q0, q1, y2, dis, b2.reshape(1, d),
                   batch.astype(jnp.int32).reshape(1, n),
                   fc_w, fc_b.reshape(1, fc_w.shape[1]))
    return out


# speedup vs baseline: 1.0120x; 1.0035x over previous
import functools

import jax
import jax.numpy as jnp
from jax import lax
from jax.experimental import pallas as pl
from jax.experimental.pallas import tpu as pltpu
from jax.experimental.pallas import tpu_sc as plsc

NC = 2
NS = 16
NW = NC * NS
LANE = 128
NUM_GRAPHS = 64


def _mesh():
    return plsc.VectorSubcoreMesh(core_axis_name="c", subcore_axis_name="s")


@functools.lru_cache(maxsize=None)
def _make_deg(npad, nblk, stripe):
    @functools.partial(
        pl.kernel,
        mesh=_mesh(),
        out_type=[
            jax.ShapeDtypeStruct((npad,), jnp.float32),
            jax.ShapeDtypeStruct((npad,), jnp.float32),
        ],
        scratch_types=[
            pltpu.VMEM((nblk, LANE), jnp.int32),
            pltpu.VMEM((LANE,), jnp.float32),
            pltpu.VMEM_SHARED((npad,), jnp.float32),
        ],
    )
    def deg_kernel(dst_hbm, zeros_hbm, out0, out1, idx_v, ones_v, acc_sh):
        c = lax.axis_index("c")
        s = lax.axis_index("s")
        wid = s * NC + c
        seg = pl.ds(s * stripe, stripe)
        pltpu.sync_copy(zeros_hbm, acc_sh.at[seg])
        pltpu.sync_copy(dst_hbm.at[wid], idx_v)
        for i in range(LANE // 16):
            ones_v[pl.ds(i * 16, 16)] = jnp.full((16,), 1.0, jnp.float32)
        plsc.subcore_barrier()

        def body(j, carry):
            pltpu.sync_copy(ones_v, acc_sh.at[idx_v.at[j]], add=True)
            return carry

        lax.fori_loop(0, nblk, body, 0)
        plsc.subcore_barrier()

        @pl.when(c == 0)
        def _():
            pltpu.sync_copy(acc_sh.at[seg], out0.at[seg])

        @pl.when(c == 1)
        def _():
            pltpu.sync_copy(acc_sh.at[seg], out1.at[seg])

    return deg_kernel


KSB = 16


@functools.lru_cache(maxsize=None)
def _make_agg(npad, nblk, stripe, d):
    assert nblk % 2 == 0 and nblk % KSB == 0 and KSB % 2 == 0
    nsb = nblk // KSB

    @functools.partial(
        pl.kernel,
        mesh=_mesh(),
        out_type=[
            jax.ShapeDtypeStruct((npad, d), jnp.float32),
            jax.ShapeDtypeStruct((npad, d), jnp.float32),
        ],
        scratch_types=[
            pltpu.VMEM((KSB, LANE), jnp.int32),
            pltpu.VMEM((KSB, LANE), jnp.int32),
            pltpu.VMEM((KSB, LANE), jnp.int32),
            pltpu.VMEM((KSB, LANE), jnp.int32),
            pltpu.VMEM((LANE, d), jnp.float32),
            pltpu.VMEM((LANE, d), jnp.float32),
            pltpu.VMEM_SHARED((npad, d), jnp.float32),
            pltpu.SemaphoreType.DMA,
            pltpu.SemaphoreType.DMA,
            pltpu.SemaphoreType.DMA,
            pltpu.SemaphoreType.DMA,
            pltpu.SemaphoreType.DMA,
        ],
    )
    def agg_kernel(y_hbm, src_hbm, dst_hbm, zrows_hbm, out0, out1,
                   si0, di0, si1, di1, rows_a, rows_b, acc_sh,
                   sem_ga, sem_gb, sem_sa, sem_sb, sem_i):
        c = lax.axis_index("c")
        s = lax.axis_index("s")
        wid = s * NC + c
        seg = pl.ds(s * stripe, stripe)
        pltpu.sync_copy(src_hbm.at[wid, pl.ds(0, KSB)], si0)
        pltpu.sync_copy(dst_hbm.at[wid, pl.ds(0, KSB)], di0)
        pltpu.sync_copy(zrows_hbm, acc_sh.at[seg])
        plsc.subcore_barrier()

        idx_bufs = [(si0, di0), (si1, di1)]
        for sb in range(nsb):
            si, di = idx_bufs[sb % 2]
            sn, dn = idx_bufs[(sb + 1) % 2]
            if sb == 0:
                pltpu.async_copy(y_hbm.at[si.at[0]], rows_a, sem_ga)
                pltpu.async_copy(y_hbm.at[si.at[1]], rows_b, sem_gb)
            if sb + 1 < nsb:
                nxt = pl.ds((sb + 1) * KSB, KSB)
                pltpu.async_copy(src_hbm.at[wid, nxt], sn, sem_i)
                pltpu.async_copy(dst_hbm.at[wid, nxt], dn, sem_i)

            def body(p, carry, si=si, di=di):
                j = p * 2
                pltpu.make_async_copy(
                    y_hbm.at[si.at[j]], rows_a, sem_ga).wait()
                pltpu.async_copy(rows_a, acc_sh.at[di.at[j]], sem_sa,
                                 add=True)
                pltpu.make_async_copy(
                    y_hbm.at[si.at[j + 1]], rows_b, sem_gb).wait()
                pltpu.async_copy(rows_b, acc_sh.at[di.at[j + 1]], sem_sb,
                                 add=True)
                pltpu.make_async_copy(
                    rows_a, acc_sh.at[di.at[j]], sem_sa).wait()
                pltpu.async_copy(y_hbm.at[si.at[j + 2]], rows_a, sem_ga)
                pltpu.make_async_copy(
                    rows_b, acc_sh.at[di.at[j + 1]], sem_sb).wait()
                pltpu.async_copy(y_hbm.at[si.at[j + 3]], rows_b, sem_gb)
                return carry

            lax.fori_loop(0, KSB // 2 - 1, body, 0)

            jl = KSB - 2
            pltpu.make_async_copy(y_hbm.at[si.at[jl]], rows_a, sem_ga).wait()
            pltpu.async_copy(rows_a, acc_sh.at[di.at[jl]], sem_sa, add=True)
            pltpu.make_async_copy(
                y_hbm.at[si.at[jl + 1]], rows_b, sem_gb).wait()
            pltpu.async_copy(rows_b, acc_sh.at[di.at[jl + 1]], sem_sb,
                             add=True)
            if sb + 1 < nsb:
                nxt = pl.ds((sb + 1) * KSB, KSB)
                pltpu.make_async_copy(
                    src_hbm.at[wid, nxt], sn, sem_i).wait()
                pltpu.make_async_copy(
                    dst_hbm.at[wid, nxt], dn, sem_i).wait()
                pltpu.make_async_copy(
                    rows_a, acc_sh.at[di.at[jl]], sem_sa).wait()
                pltpu.async_copy(y_hbm.at[sn.at[0]], rows_a, sem_ga)
                pltpu.make_async_copy(
                    rows_b, acc_sh.at[di.at[jl + 1]], sem_sb).wait()
                pltpu.async_copy(y_hbm.at[sn.at[1]], rows_b, sem_gb)
            else:
                pltpu.make_async_copy(
                    rows_a, acc_sh.at[di.at[jl]], sem_sa).wait()
                pltpu.make_async_copy(
                    rows_b, acc_sh.at[di.at[jl + 1]], sem_sb).wait()
        plsc.subcore_barrier()

        @pl.when(c == 0)
        def _():
            pltpu.sync_copy(acc_sh.at[seg], out0.at[seg])

        @pl.when(c == 1)
        def _():
            pltpu.sync_copy(acc_sh.at[seg], out1.at[seg])

    return agg_kernel


def _prep_tc(x, w1, d0, d1):
    n = x.shape[0]
    d = w1.shape[1]

    def body(x_ref, w_ref, d0_ref, d1_ref, y_ref, dis_ref):
        deg = d0_ref[...][:n] + d1_ref[...][:n] + 1.0
        dis = lax.rsqrt(deg)
        dis_ref[...] = dis
        xw = jnp.dot(x_ref[...], w_ref[...], preferred_element_type=jnp.float32)
        y_ref[...] = xw * dis

    return pl.pallas_call(
        body,
        out_shape=[
            jax.ShapeDtypeStruct((n, d), jnp.float32),
            jax.ShapeDtypeStruct((n, 1), jnp.float32),
        ],
    )(x, w1, d0, d1)


def _mid_tc(p0, p1, y1, dis, b1, w2):
    n, d = y1.shape

    def body(p0_ref, p1_ref, y1_ref, dis_ref, b_ref, w_ref, y2_ref):
        acc = p0_ref[...][:n] + p1_ref[...][:n] + y1_ref[...]
        h = jnp.maximum(dis_ref[...] * acc + b_ref[...], 0.0)
        hw = jnp.dot(h, w_ref[...], preferred_element_type=jnp.float32)
        y2_ref[...] = hw * dis_ref[...]

    return pl.pallas_call(
        body,
        out_shape=jax.ShapeDtypeStruct((n, d), jnp.float32),
    )(p0, p1, y1, dis, b1, w2)


def _head_tc(q0, q1, y2, dis, b2, batch_row, fc_w, fc_b):
    n, d = y2.shape
    d_out = fc_w.shape[1]

    def body(q0_ref, q1_ref, y2_ref, dis_ref, b_ref, batch_ref, fw_ref,
             fb_ref, out_ref):
        acc = q0_ref[...][:n] + q1_ref[...][:n] + y2_ref[...]
        h = jnp.maximum(dis_ref[...] * acc + b_ref[...], 0.0)
        gids = lax.broadcasted_iota(jnp.int32, (NUM_GRAPHS, n), 0)
        m = (gids == batch_ref[...]).astype(jnp.float32)
        sums = jnp.dot(m, h, preferred_element_type=jnp.float32)
        counts = jnp.sum(m, axis=1, keepdims=True)
        pooled = sums / jnp.maximum(counts, 1.0)
        out_ref[...] = (
            jnp.dot(pooled, fw_ref[...], preferred_element_type=jnp.float32)
            + fb_ref[...]
        )

    return pl.pallas_call(
        body,
        out_shape=jax.ShapeDtypeStruct((NUM_GRAPHS, d_out), jnp.float32),
    )(q0, q1, y2, dis, b2, batch_row, fc_w, fc_b)


def kernel(x, edge_index, batch, W1, b1, W2, b2, fc_w, fc_b):
    n, _ = x.shape
    d = W1.shape[1]
    e = edge_index.shape[1]

    npad = -(-(n + 8) // (NS * LANE)) * (NS * LANE)
    stripe = npad // NS
    nblk = -(-e // (NW * LANE))
    nblk = -(-nblk // KSB) * KSB
    pade = NW * nblk * LANE

    ei = edge_index.astype(jnp.int32)
    pad = pade - e
    pad_ar = jnp.arange(pad, dtype=jnp.int32)
    src_p = jnp.concatenate(
        [ei[0], pad_ar % n]).reshape(NW, nblk, LANE)
    dst_p = jnp.concatenate(
        [ei[1], n + pad_ar % (npad - n)]).reshape(NW, nblk, LANE)

    z1 = jnp.zeros((stripe,), jnp.float32)
    z2 = jnp.zeros((stripe, d), jnp.float32)

    deg0, deg1 = _make_deg(npad, nblk, stripe)(dst_p, z1)
    y1, dis = _prep_tc(x, W1, deg0.reshape(npad, 1), deg1.reshape(npad, 1))

    agg = _make_agg(npad, nblk, stripe, d)
    p0, p1 = agg(y1, src_p, dst_p, z2)
    y2 = _mid_tc(p0, p1, y1, dis, b1.reshape(1, d), W2)
    q0, q1 = agg(y2, src_p, dst_p, z2)

    out = _head_tc(q0, q1, y2, dis, b2.reshape(1, d),
                   batch.astype(jnp.int32).reshape(1, n),
                   fc_w, fc_b.reshape(1, fc_w.shape[1]))
    return out
